# baseline (speedup 1.0000x reference)
import jax
import jax.numpy as jnp
from jax import lax
from jax.experimental import pallas as pl
from jax.experimental.pallas import tpu as pltpu
from jax.experimental.pallas import tpu_sc as plsc

VOCAB = 1000000
EMBED = 64
BATCH = 1024
CTX = 5

_NC = 2
_NS = 16
_NW = _NC * _NS

_TILE = 128
_HALF = 64


def _emit_job(tbl, idx_hbm, idx_base, ncols, dst_sink,
              idxbuf, staging, outbuf, sems):
    pltpu.sync_copy(idx_hbm.at[pl.ds(idx_base, ncols)],
                    idxbuf.at[pl.ds(0, ncols)])
    row_idx = [lax.iota(jnp.int32, 16) + 16 * m for m in range(4)]

    def _select(slot, r, col):
        lane = jnp.full((16,), r & 127, jnp.int32)
        dst = jnp.full((16,), col, jnp.int32)
        for m in range(4):
            v = plsc.load_gather(staging[slot], [row_idx[m], lane])
            plsc.store_scatter(outbuf, [row_idx[m], dst], v)

    def chunk(k, vec_prev):
        vec = idxbuf[pl.ds(k * 16, 16)]
        for j in range(16):
            col = k * 16 + j
            slot = j % 4
            if j < 4:
                def drain(vp=vec_prev, s=slot, jj=j, c=col):
                    pltpu.make_async_copy(
                        tbl.at[:, pl.ds(0, _TILE)], staging[s],
                        sems[s]).wait()
                    _select(s, vp[12 + jj], c - 4)
                pl.when(k > 0)(drain)
            else:
                pltpu.make_async_copy(
                    tbl.at[:, pl.ds(0, _TILE)], staging[slot],
                    sems[slot]).wait()
                _select(slot, vec[j - 4], col - 4)
            blk = pl.multiple_of((vec[j] >> 7) * _TILE, _TILE)
            pltpu.async_copy(tbl.at[:, pl.ds(blk, _TILE)], staging[slot],
                             sems[slot])
        return vec

    vec_last = lax.fori_loop(0, ncols // 16, chunk,
                             jnp.zeros((16,), jnp.int32))
    for j in range(4):
        pltpu.make_async_copy(tbl.at[:, pl.ds(0, _TILE)], staging[j],
                              sems[j]).wait()
        _select(j, vec_last[12 + j], ncols - 4 + j)
    dst_sink()


def _gather_a_body(wtT_hbm, wcT_hbm, catidx_hbm, out_hbm,
                   idxbuf, s0, s1, s2, s3, outbuf, shared, m0, m1, m2, m3):
    wid = lax.axis_index("s") * _NC + lax.axis_index("c")
    staging = [s0, s1, s2, s3]
    sems = [m0, m1, m2, m3]
    t16 = wid & 15
    half = wid >> 4
    idx_base = t16 * _TILE + half * _HALF

    def spmem_sink():
        pltpu.sync_copy(outbuf, shared.at[t16, half])

    def job(tbl):
        def run():
            _emit_job(tbl, catidx_hbm, idx_base, _HALF, spmem_sink,
                      idxbuf, staging, outbuf, sems)
        return run

    pl.when(t16 < 8)(job(wtT_hbm))
    pl.when(t16 >= 8)(job(wcT_hbm))
    plsc.subcore_barrier()

    def assemble():
        pltpu.sync_copy(shared.at[wid, 0], outbuf)
        pltpu.sync_copy(shared.at[wid, 1], s0)
        for row in range(EMBED):
            for m in range(_HALF // 16):
                outbuf[row, pl.ds(_HALF + 16 * m, 16)] = (
                    s0[row, pl.ds(16 * m, 16)])
        off = pl.multiple_of(wid * _TILE, _TILE)
        pltpu.sync_copy(outbuf, out_hbm.at[:, pl.ds(off, _TILE)])

    pl.when(wid < 16)(assemble)


_SCRATCH_A = [
    pltpu.VMEM((_TILE,), jnp.int32),
    pltpu.VMEM((EMBED, _TILE), jnp.float32),
    pltpu.VMEM((EMBED, _TILE), jnp.float32),
    pltpu.VMEM((EMBED, _TILE), jnp.float32),
    pltpu.VMEM((EMBED, _TILE), jnp.float32),
    pltpu.VMEM((EMBED, _TILE), jnp.float32),
    pltpu.VMEM_SHARED((16, 2, EMBED, _TILE), jnp.float32),
    pltpu.SemaphoreType.DMA,
    pltpu.SemaphoreType.DMA,
    pltpu.SemaphoreType.DMA,
    pltpu.SemaphoreType.DMA,
]

_gather_a = pl.kernel(
    _gather_a_body,
    out_type=jax.ShapeDtypeStruct((EMBED, 2 * BATCH), jnp.float32),
    mesh=plsc.VectorSubcoreMesh(core_axis_name="c", subcore_axis_name="s"),
    compiler_params=pltpu.CompilerParams(needs_layout_passes=False),
    scratch_types=_SCRATCH_A,
)


def _gather_b_body(wcT_hbm, cidx_hbm, out_hbm,
                   idxbuf, s0, s1, s2, s3, outbuf, m0, m1, m2, m3):
    wid = lax.axis_index("s") * _NC + lax.axis_index("c")
    staging = [s0, s1, s2, s3]
    sems = [m0, m1, m2, m3]

    def sink():
        off = pl.multiple_of(wid * _TILE, _TILE)
        pltpu.sync_copy(outbuf, out_hbm.at[:, pl.ds(off, _TILE)])

    _emit_job(wcT_hbm, cidx_hbm, wid * _TILE, _TILE, sink,
              idxbuf, staging, outbuf, sems)


_gather_b = pl.kernel(
    _gather_b_body,
    out_type=jax.ShapeDtypeStruct((EMBED, 4 * BATCH), jnp.float32),
    mesh=plsc.VectorSubcoreMesh(core_axis_name="c", subcore_axis_name="s"),
    compiler_params=pltpu.CompilerParams(needs_layout_passes=False),
    scratch_types=_SCRATCH_A[:6] + _SCRATCH_A[7:],
)


def _mm1_body(ab_ref, o_ref):
    o_ref[0] = lax.dot_general(
        ab_ref[:, : BATCH], ab_ref[:, BATCH:],
        dimension_numbers=(((0,), (0,)), ((), ())),
        preferred_element_type=jnp.float32,
    )


def _mm2_body(a_ref, b_ref, prev_ref, o_ref):
    del prev_ref
    o_ref[0] = lax.dot_general(
        a_ref[...], b_ref[...],
        dimension_numbers=(((0,), (0,)), ((), ())),
        preferred_element_type=jnp.float32,
    )


_BN = 512


def kernel(target, context, W_target, W_context):
    tidx = jnp.asarray(target, jnp.int32).reshape(BATCH)
    ctx32 = jnp.asarray(context, jnp.int32)
    catidx = jnp.concatenate([tidx, ctx32[:, 0]])
    cidx2 = ctx32[:, 1:].T.reshape(BATCH * (CTX - 1))
    wtT = W_target.T
    wcT = W_context.T
    embT_a = _gather_a(wtT, wcT, catidx)
    embT_b = _gather_b(wcT, cidx2)
    out5a = pl.pallas_call(
        _mm1_body,
        grid=(1,),
        in_specs=[pl.BlockSpec((EMBED, 2 * BATCH), lambda j: (0, 0))],
        out_specs=pl.BlockSpec((1, BATCH, BATCH), lambda j: (0, 0, 0)),
        out_shape=jax.ShapeDtypeStruct((CTX, BATCH, BATCH), jnp.float32),
    )(embT_a)
    nj = BATCH // _BN
    out5 = pl.pallas_call(
        _mm2_body,
        grid=(CTX - 1, nj),
        in_specs=[
            pl.BlockSpec((EMBED, BATCH), lambda c, j: (0, 0)),
            pl.BlockSpec((EMBED, _BN), lambda c, j: (0, c * nj + j)),
            pl.BlockSpec(memory_space=pl.ANY),
        ],
        out_specs=pl.BlockSpec((1, BATCH, _BN), lambda c, j: (c + 1, 0, j)),
        out_shape=jax.ShapeDtypeStruct((CTX, BATCH, BATCH), jnp.float32),
        input_output_aliases={2: 0},
    )(embT_a, embT_b, out5a)
    return jnp.transpose(out5, (1, 2, 0))


# speedup vs baseline: 1.0343x; 1.0343x over previous
import jax
import jax.numpy as jnp
from jax import lax
from jax.experimental import pallas as pl
from jax.experimental.pallas import tpu as pltpu
from jax.experimental.pallas import tpu_sc as plsc

VOCAB = 1000000
EMBED = 64
BATCH = 1024
CTX = 5

_NC = 2
_NS = 16
_NW = _NC * _NS

_TILE = 128
_HALF = 64
_TGT_TILES = BATCH // _TILE
_CTX_TILES = BATCH * CTX // _TILE
_SHARED_TILES = 16


def _emit_job(tbl, idx_hbm, idx_base, ncols, dst_sink,
              idxbuf, staging, outbuf, sems):
    pltpu.sync_copy(idx_hbm.at[pl.ds(idx_base, ncols)],
                    idxbuf.at[pl.ds(0, ncols)])
    row_idx = [lax.iota(jnp.int32, 16) + 16 * m for m in range(4)]

    def _select(slot, r, col):
        lane = jnp.full((16,), r & 127, jnp.int32)
        dst = jnp.full((16,), col, jnp.int32)
        for m in range(4):
            v = plsc.load_gather(staging[slot], [row_idx[m], lane])
            plsc.store_scatter(outbuf, [row_idx[m], dst], v)

    def chunk(k, vec_prev):
        vec = idxbuf[pl.ds(k * 16, 16)]
        for j in range(16):
            col = k * 16 + j
            slot = j % 4
            if j < 4:
                def drain(vp=vec_prev, s=slot, jj=j, c=col):
                    pltpu.make_async_copy(
                        tbl.at[:, pl.ds(0, _TILE)], staging[s],
                        sems[s]).wait()
                    _select(s, vp[12 + jj], c - 4)
                pl.when(k > 0)(drain)
            else:
                pltpu.make_async_copy(
                    tbl.at[:, pl.ds(0, _TILE)], staging[slot],
                    sems[slot]).wait()
                _select(slot, vec[j - 4], col - 4)
            blk = pl.multiple_of((vec[j] >> 7) * _TILE, _TILE)
            pltpu.async_copy(tbl.at[:, pl.ds(blk, _TILE)], staging[slot],
                             sems[slot])
        return vec

    vec_last = lax.fori_loop(0, ncols // 16, chunk,
                             jnp.zeros((16,), jnp.int32))
    for j in range(4):
        pltpu.make_async_copy(tbl.at[:, pl.ds(0, _TILE)], staging[j],
                              sems[j]).wait()
        _select(j, vec_last[12 + j], ncols - 4 + j)
    dst_sink()


def _gather_body(wtT_hbm, wcT_hbm, tidx_hbm, cidx_hbm, out_t_hbm, out_c_hbm,
                 idxbuf, s0, s1, s2, s3, outbuf, shared, m0, m1, m2, m3):
    wid = lax.axis_index("s") * _NC + lax.axis_index("c")
    staging = [s0, s1, s2, s3]
    sems = [m0, m1, m2, m3]

    def hbm_sink(out_hbm, col_base):
        def sink():
            off = pl.multiple_of(col_base, _TILE)
            pltpu.sync_copy(outbuf, out_hbm.at[:, pl.ds(off, _TILE)])
        return sink

    def job_t(tile):
        _emit_job(wtT_hbm, tidx_hbm, tile * _TILE, _TILE,
                  hbm_sink(out_t_hbm, tile * _TILE),
                  idxbuf, staging, outbuf, sems)

    def job_c(tile):
        _emit_job(wcT_hbm, cidx_hbm, tile * _TILE, _TILE,
                  hbm_sink(out_c_hbm, tile * _TILE),
                  idxbuf, staging, outbuf, sems)

    pl.when(wid < _TGT_TILES)(lambda: job_t(wid))
    pl.when(wid >= _TGT_TILES)(lambda: job_c(wid - _TGT_TILES))

    t16 = wid & 15
    half = wid >> 4
    idx_base = (_CTX_TILES - _SHARED_TILES + t16) * _TILE + half * _HALF

    def spmem_sink():
        pltpu.sync_copy(outbuf, shared.at[t16, half])

    _emit_job(wcT_hbm, cidx_hbm, idx_base, _HALF, spmem_sink,
              idxbuf, staging, outbuf, sems)
    plsc.subcore_barrier()

    def assemble():
        tile = _CTX_TILES - _SHARED_TILES + wid
        pltpu.sync_copy(shared.at[wid, 0], outbuf)
        pltpu.sync_copy(shared.at[wid, 1], s0)
        for row in range(EMBED):
            for m in range(_HALF // 16):
                outbuf[row, pl.ds(_HALF + 16 * m, 16)] = (
                    s0[row, pl.ds(16 * m, 16)])
        off = pl.multiple_of(tile * _TILE, _TILE)
        pltpu.sync_copy(outbuf, out_c_hbm.at[:, pl.ds(off, _TILE)])

    pl.when(wid < _SHARED_TILES)(assemble)


_gather = pl.kernel(
    _gather_body,
    out_type=(
        jax.ShapeDtypeStruct((EMBED, BATCH), jnp.float32),
        jax.ShapeDtypeStruct((EMBED, BATCH * CTX), jnp.float32),
    ),
    mesh=plsc.VectorSubcoreMesh(core_axis_name="c", subcore_axis_name="s"),
    compiler_params=pltpu.CompilerParams(needs_layout_passes=False),
    scratch_types=[
        pltpu.VMEM((_TILE,), jnp.int32),
        pltpu.VMEM((EMBED, _TILE), jnp.float32),
        pltpu.VMEM((EMBED, _TILE), jnp.float32),
        pltpu.VMEM((EMBED, _TILE), jnp.float32),
        pltpu.VMEM((EMBED, _TILE), jnp.float32),
        pltpu.VMEM((EMBED, _TILE), jnp.float32),
        pltpu.VMEM_SHARED((_SHARED_TILES, 2, EMBED, _TILE), jnp.float32),
        pltpu.SemaphoreType.DMA,
        pltpu.SemaphoreType.DMA,
        pltpu.SemaphoreType.DMA,
        pltpu.SemaphoreType.DMA,
    ],
)


def _mm_body(a_ref, b_ref, o_ref):
    o_ref[0] = lax.dot_general(
        a_ref[...], b_ref[...],
        dimension_numbers=(((0,), (0,)), ((), ())),
        preferred_element_type=jnp.float32,
    )


_BN = 1024


def kernel(target, context, W_target, W_context):
    tidx = jnp.asarray(target, jnp.int32).reshape(BATCH)
    cidx = jnp.asarray(context, jnp.int32).T.reshape(BATCH * CTX)
    embT_t, embT_c = _gather(W_target.T, W_context.T, tidx, cidx)
    out5 = pl.pallas_call(
        _mm_body,
        grid=(CTX, BATCH // _BN),
        in_specs=[
            pl.BlockSpec((EMBED, BATCH), lambda c, j: (0, 0)),
            pl.BlockSpec((EMBED, _BN),
                         lambda c, j: (0, c * (BATCH // _BN) + j)),
        ],
        out_specs=pl.BlockSpec((1, BATCH, _BN), lambda c, j: (c, 0, j)),
        out_shape=jax.ShapeDtypeStruct((CTX, BATCH, BATCH), jnp.float32),
    )(embT_t, embT_c)
    return jnp.transpose(out5, (1, 2, 0))


# speedup vs baseline: 1.0348x; 1.0005x over previous
import jax
import jax.numpy as jnp
from jax import lax
from jax.experimental import pallas as pl
from jax.experimental.pallas import tpu as pltpu
from jax.experimental.pallas import tpu_sc as plsc

VOCAB = 1000000
EMBED = 64
BATCH = 1024
CTX = 5

_NC = 2
_NS = 16
_NW = _NC * _NS

_TILE = 128
_HALF = 64
_TGT_TILES = BATCH // _TILE
_CTX_TILES = BATCH * CTX // _TILE
_SHARED_TILES = 16


def _emit_job(tbl, idx_hbm, idx_base, ncols, dst_sink,
              idxbuf, staging, outbuf, sems):
    pltpu.sync_copy(idx_hbm.at[pl.ds(idx_base, ncols)],
                    idxbuf.at[pl.ds(0, ncols)])
    row_idx = [lax.iota(jnp.int32, 16) + 16 * m for m in range(4)]

    def _select(slot, r, col):
        l = r - jnp.minimum((r >> 7) * _TILE, VOCAB - _TILE)
        lane = jnp.full((16,), l, jnp.int32)
        dst = jnp.full((16,), col, jnp.int32)
        for m in range(4):
            v = plsc.load_gather(staging[slot], [row_idx[m], lane])
            plsc.store_scatter(outbuf, [row_idx[m], dst], v)

    def chunk(k, vec_prev):
        vec = idxbuf[pl.ds(k * 16, 16)]
        for j in range(16):
            col = k * 16 + j
            slot = j % 4
            if j < 4:
                def drain(vp=vec_prev, s=slot, jj=j, c=col):
                    pltpu.make_async_copy(
                        tbl.at[:, pl.ds(0, _TILE)], staging[s],
                        sems[s]).wait()
                    _select(s, vp[12 + jj], c - 4)
                pl.when(k > 0)(drain)
            else:
                pltpu.make_async_copy(
                    tbl.at[:, pl.ds(0, _TILE)], staging[slot],
                    sems[slot]).wait()
                _select(slot, vec[j - 4], col - 4)
            blk = pl.multiple_of(
                jnp.minimum((vec[j] >> 7) * _TILE, VOCAB - _TILE), _TILE)
            pltpu.async_copy(tbl.at[:, pl.ds(blk, _TILE)], staging[slot],
                             sems[slot])
        return vec

    vec_last = lax.fori_loop(0, ncols // 16, chunk,
                             jnp.zeros((16,), jnp.int32))
    for j in range(4):
        pltpu.make_async_copy(tbl.at[:, pl.ds(0, _TILE)], staging[j],
                              sems[j]).wait()
        _select(j, vec_last[12 + j], ncols - 4 + j)
    dst_sink()


def _gather_body(wtT_hbm, wcT_hbm, tidx_hbm, cidx_hbm, out_t_hbm, out_c_hbm,
                 idxbuf, s0, s1, s2, s3, outbuf, shared, m0, m1, m2, m3):
    wid = lax.axis_index("s") * _NC + lax.axis_index("c")
    staging = [s0, s1, s2, s3]
    sems = [m0, m1, m2, m3]

    def hbm_sink(out_hbm, col_base):
        def sink():
            off = pl.multiple_of(col_base, _TILE)
            pltpu.sync_copy(outbuf, out_hbm.at[:, pl.ds(off, _TILE)])
        return sink

    def job_t(tile):
        _emit_job(wtT_hbm, tidx_hbm, tile * _TILE, _TILE,
                  hbm_sink(out_t_hbm, tile * _TILE),
                  idxbuf, staging, outbuf, sems)

    def job_c(tile):
        _emit_job(wcT_hbm, cidx_hbm, tile * _TILE, _TILE,
                  hbm_sink(out_c_hbm, tile * _TILE),
                  idxbuf, staging, outbuf, sems)

    pl.when(wid < _TGT_TILES)(lambda: job_t(wid))
    pl.when(wid >= _TGT_TILES)(lambda: job_c(wid - _TGT_TILES))

    t16 = wid & 15
    half = wid >> 4
    idx_base = (_CTX_TILES - _SHARED_TILES + t16) * _TILE + half * _HALF

    def spmem_sink():
        pltpu.sync_copy(outbuf, shared.at[t16, half])

    _emit_job(wcT_hbm, cidx_hbm, idx_base, _HALF, spmem_sink,
              idxbuf, staging, outbuf, sems)
    plsc.subcore_barrier()

    def assemble():
        tile = _CTX_TILES - _SHARED_TILES + wid
        pltpu.sync_copy(shared.at[wid, 0], outbuf)
        pltpu.sync_copy(shared.at[wid, 1], s0)
        for row in range(EMBED):
            for m in range(_HALF // 16):
                outbuf[row, pl.ds(_HALF + 16 * m, 16)] = (
                    s0[row, pl.ds(16 * m, 16)])
        off = pl.multiple_of(tile * _TILE, _TILE)
        pltpu.sync_copy(outbuf, out_c_hbm.at[:, pl.ds(off, _TILE)])

    pl.when(wid < _SHARED_TILES)(assemble)


_gather = pl.kernel(
    _gather_body,
    out_type=(
        jax.ShapeDtypeStruct((EMBED, BATCH), jnp.float32),
        jax.ShapeDtypeStruct((EMBED, BATCH * CTX), jnp.float32),
    ),
    mesh=plsc.VectorSubcoreMesh(core_axis_name="c", subcore_axis_name="s"),
    compiler_params=pltpu.CompilerParams(needs_layout_passes=False),
    scratch_types=[
        pltpu.VMEM((_TILE,), jnp.int32),
        pltpu.VMEM((EMBED, _TILE), jnp.float32),
        pltpu.VMEM((EMBED, _TILE), jnp.float32),
        pltpu.VMEM((EMBED, _TILE), jnp.float32),
        pltpu.VMEM((EMBED, _TILE), jnp.float32),
        pltpu.VMEM((EMBED, _TILE), jnp.float32),
        pltpu.VMEM_SHARED((_SHARED_TILES, 2, EMBED, _TILE), jnp.float32),
        pltpu.SemaphoreType.DMA,
        pltpu.SemaphoreType.DMA,
        pltpu.SemaphoreType.DMA,
        pltpu.SemaphoreType.DMA,
    ],
)


def _mm_body(a_ref, b_ref, o_ref):
    o_ref[0] = lax.dot_general(
        a_ref[...], b_ref[...],
        dimension_numbers=(((0,), (0,)), ((), ())),
        preferred_element_type=jnp.float32,
    )


_BN = 1024


def kernel(target, context, W_target, W_context):
    tidx = jnp.asarray(target, jnp.int32).reshape(BATCH)
    cidx = jnp.asarray(context, jnp.int32).T.reshape(BATCH * CTX)
    embT_t, embT_c = _gather(W_target.T, W_context.T, tidx, cidx)
    out5 = pl.pallas_call(
        _mm_body,
        grid=(CTX, BATCH // _BN),
        in_specs=[
            pl.BlockSpec((EMBED, BATCH), lambda c, j: (0, 0)),
            pl.BlockSpec((EMBED, _BN),
                         lambda c, j: (0, c * (BATCH // _BN) + j)),
        ],
        out_specs=pl.BlockSpec((1, BATCH, _BN), lambda c, j: (c, 0, j)),
        out_shape=jax.ShapeDtypeStruct((CTX, BATCH, BATCH), jnp.float32),
    )(embT_t, embT_c)
    return jnp.transpose(out5, (1, 2, 0))
